# baseline (speedup 1.0000x reference)
import functools

import jax
import jax.numpy as jnp
from jax import lax
from jax.experimental import pallas as pl
from jax.experimental.pallas import tpu as pltpu

BATCH = 16384
CODE = 64


def _tc_body(idx_hbm, ws_hbm, wa_hbm, out_s, out_a, idx_sm, sem_s, sem_a, sem_i):
    pltpu.make_async_copy(idx_hbm, idx_sm, sem_i).start()
    pltpu.make_async_copy(idx_hbm, idx_sm, sem_i).wait()

    def body(i, c):
        r = idx_sm[i]
        pltpu.make_async_copy(
            ws_hbm.at[pl.ds(r, 1)], out_s.at[pl.ds(i, 1)], sem_s).start()
        pltpu.make_async_copy(
            wa_hbm.at[pl.ds(r, 1)], out_a.at[pl.ds(i, 1)], sem_a).start()
        return c

    lax.fori_loop(0, BATCH, body, 0, unroll=8)
    pltpu.make_async_copy(ws_hbm.at[pl.ds(0, BATCH)], out_s, sem_s).wait()
    pltpu.make_async_copy(wa_hbm.at[pl.ds(0, BATCH)], out_a, sem_a).wait()


_tc_gather = pl.pallas_call(
    _tc_body,
    out_shape=(
        jax.ShapeDtypeStruct((BATCH, CODE), jnp.float32),
        jax.ShapeDtypeStruct((BATCH, CODE), jnp.float32),
    ),
    in_specs=[
        pl.BlockSpec(memory_space=pl.ANY),
        pl.BlockSpec(memory_space=pl.ANY),
        pl.BlockSpec(memory_space=pl.ANY),
    ],
    out_specs=(
        pl.BlockSpec(memory_space=pl.ANY),
        pl.BlockSpec(memory_space=pl.ANY),
    ),
    scratch_shapes=[
        pltpu.SMEM((BATCH,), jnp.int32),
        pltpu.SemaphoreType.DMA,
        pltpu.SemaphoreType.DMA,
        pltpu.SemaphoreType.DMA,
    ],
)


def kernel(instance_ids, W_shape, W_appearance):
    idx = instance_ids.astype(jnp.int32)
    return _tc_gather(idx, W_shape, W_appearance)
